# Initial kernel scaffold; baseline (speedup 1.0000x reference)
import functools

import jax
import jax.numpy as jnp
from jax import lax
from jax.experimental import pallas as pl
from jax.experimental.pallas import tpu as pltpu
from jax.experimental.pallas import tpu_sc as plsc

N = 10000
D = 128
EPS = 1e-5
L = 16
NC, NS = 2, 16
NW = NC * NS
K = 128
E = 320000
CPT = -(-E // (NW * K))
EPAD = CPT * NW * K
NPAD = 10240
RPT = NPAD // NS

_mesh = plsc.VectorSubcoreMesh(core_axis_name="c", subcore_axis_name="s")


@functools.partial(
    pl.kernel,
    out_type=jax.ShapeDtypeStruct((NC, NPAD, L), jnp.float32),
    mesh=_mesh,
    scratch_types=[
        pltpu.VMEM((CPT, K), jnp.int32),
        pltpu.VMEM((K, L), jnp.float32),
        pltpu.VMEM_SHARED((NPAD, L), jnp.float32),
    ],
)
def _deg_kernel(dst_hbm, ones_hbm, zeros16_hbm, out_hbm, dst_v, ones_v, acc_sh):
    cid = lax.axis_index("c")
    sid = lax.axis_index("s")
    wid = sid * NC + cid
    pltpu.sync_copy(dst_hbm.at[wid], dst_v)
    pltpu.sync_copy(ones_hbm, ones_v)
    pltpu.sync_copy(zeros16_hbm, acc_sh.at[pl.ds(sid * RPT, RPT)])
    plsc.subcore_barrier()

    def body(j, carry):
        pltpu.sync_copy(ones_v, acc_sh.at[dst_v.at[j]], add=True)
        return carry

    lax.fori_loop(0, CPT, body, 0)
    plsc.subcore_barrier()
    pltpu.sync_copy(acc_sh.at[pl.ds(sid * RPT, RPT)],
                    out_hbm.at[cid, pl.ds(sid * RPT, RPT)])


def _make_edge_kernel(with_c):
    out_type = [jax.ShapeDtypeStruct((NC, NPAD, D), jnp.float32)]
    scratch = [
        pltpu.VMEM((CPT, K), jnp.int32),
        pltpu.VMEM((CPT, K), jnp.int32),
        pltpu.VMEM((K, D), jnp.float32),
        pltpu.VMEM_SHARED((NPAD, D), jnp.float32),
        pltpu.SemaphoreType.DMA,
    ]
    if with_c:
        out_type.append(jax.ShapeDtypeStruct((NC, NPAD, L), jnp.float32))
        scratch += [
            pltpu.VMEM((K, L), jnp.float32),
            pltpu.VMEM_SHARED((NPAD, L), jnp.float32),
            pltpu.SemaphoreType.DMA,
        ]

    def kernel_body(*refs):
        if with_c:
            (src_hbm, dst_hbm, u_hbm, z128_hbm, dinv16_hbm, z16_hbm,
             acc_out, c_out,
             src_v, dst_v, rows_v, acc_sh, sem, drows_v, c_sh, sem2) = refs
        else:
            (src_hbm, dst_hbm, u_hbm, z128_hbm,
             acc_out,
             src_v, dst_v, rows_v, acc_sh, sem) = refs
        cid = lax.axis_index("c")
        sid = lax.axis_index("s")
        wid = sid * NC + cid
        pltpu.sync_copy(src_hbm.at[wid], src_v)
        pltpu.sync_copy(dst_hbm.at[wid], dst_v)
        pltpu.sync_copy(z128_hbm, acc_sh.at[pl.ds(sid * RPT, RPT)])
        if with_c:
            pltpu.sync_copy(z16_hbm, c_sh.at[pl.ds(sid * RPT, RPT)])
        plsc.subcore_barrier()

        def body(j, carry):
            pltpu.async_copy(u_hbm.at[src_v.at[j]], rows_v, sem).wait()
            pltpu.sync_copy(rows_v, acc_sh.at[dst_v.at[j]], add=True)
            if with_c:
                pltpu.async_copy(dinv16_hbm.at[dst_v.at[j]], drows_v, sem2).wait()
                pltpu.sync_copy(drows_v, c_sh.at[src_v.at[j]], add=True)
            return carry

        lax.fori_loop(0, CPT, body, 0)
        plsc.subcore_barrier()
        pltpu.sync_copy(acc_sh.at[pl.ds(sid * RPT, RPT)],
                        acc_out.at[cid, pl.ds(sid * RPT, RPT)])
        if with_c:
            pltpu.sync_copy(c_sh.at[pl.ds(sid * RPT, RPT)],
                            c_out.at[cid, pl.ds(sid * RPT, RPT)])

    return pl.kernel(
        kernel_body,
        out_type=tuple(out_type) if with_c else out_type[0],
        mesh=_mesh,
        scratch_types=scratch,
    )


_edge_kernel_c = _make_edge_kernel(True)
_edge_kernel = _make_edge_kernel(False)


def _t1_body(degp_ref, x_ref, w1_ref, u1_ref, dinv16_ref):
    deg16 = degp_ref[0] + degp_ref[1] + 1.0
    dinv16 = lax.rsqrt(deg16)
    dinv16_ref[...] = dinv16
    dinv = dinv16[:N, :1]
    h = jnp.dot(x_ref[...], w1_ref[...], preferred_element_type=jnp.float32)
    u1_ref[:N] = h * dinv
    u1_ref[N:] = jnp.zeros((NPAD - N, D), jnp.float32)


def _t2_body(acc_ref, u_ref, dinv16_ref, b_ref, g_ref, be_ref, w2_ref, out_ref):
    dinv = dinv16_ref[:N, :1]
    z = (acc_ref[0, :N] + acc_ref[1, :N] + u_ref[:N]) * dinv + b_ref[...]
    m = jnp.mean(z, axis=0, keepdims=True)
    zc = z - m
    v = jnp.mean(zc * zc, axis=0, keepdims=True)
    h = jnp.maximum(zc * lax.rsqrt(v + EPS) * g_ref[...] + be_ref[...], 0.0)
    u2 = jnp.dot(h, w2_ref[...], preferred_element_type=jnp.float32) * dinv
    out_ref[:N] = u2
    out_ref[N:] = jnp.zeros((NPAD - N, D), jnp.float32)


def _t3_body(acc_ref, u_ref, dinv16_ref, c_ref, b2_ref, g2_ref, be2_ref,
             w3_ref, b3_ref, out_ref):
    dinv = dinv16_ref[:N, :1]
    z = (acc_ref[0, :N] + acc_ref[1, :N] + u_ref[:N]) * dinv + b2_ref[...]
    m = jnp.mean(z, axis=0, keepdims=True)
    zc = z - m
    v = jnp.mean(zc * zc, axis=0, keepdims=True)
    h2 = jnp.maximum(zc * lax.rsqrt(v + EPS) * g2_ref[...] + be2_ref[...], 0.0)
    c = c_ref[0, :N, :1] + c_ref[1, :N, :1]
    w = dinv * (c + dinv)
    s = jnp.sum(h2 * w, axis=0, keepdims=True)
    out_ref[...] = (jnp.dot(s, w3_ref[...],
                            preferred_element_type=jnp.float32) * (1.0 / N)
                    + b3_ref[...])


def kernel(x, edge_index, W1, b1, g1, be1, W2, b2, g2, be2, W3, b3):
    src = edge_index[0]
    dst = edge_index[1]
    padv = jnp.full((EPAD - E,), N, jnp.int32)
    srcp = jnp.concatenate([src, padv]).reshape(NW, CPT, K)
    dstp = jnp.concatenate([dst, padv]).reshape(NW, CPT, K)

    ones_kl = jnp.ones((K, L), jnp.float32)
    zeros16 = jnp.zeros((RPT, L), jnp.float32)
    zeros128 = jnp.zeros((RPT, D), jnp.float32)

    deg_parts = _deg_kernel(dstp, ones_kl, zeros16)

    u1, dinv16 = pl.pallas_call(
        _t1_body,
        out_shape=(jax.ShapeDtypeStruct((NPAD, D), jnp.float32),
                   jax.ShapeDtypeStruct((NPAD, L), jnp.float32)),
    )(deg_parts, x, W1)

    acc1, c_parts = _edge_kernel_c(srcp, dstp, u1, zeros128, dinv16, zeros16)

    u2 = pl.pallas_call(
        _t2_body,
        out_shape=jax.ShapeDtypeStruct((NPAD, D), jnp.float32),
    )(acc1, u1, dinv16, b1.reshape(1, D), g1.reshape(1, D),
      be1.reshape(1, D), W2)

    acc2 = _edge_kernel(srcp, dstp, u2, zeros128)

    out = pl.pallas_call(
        _t3_body,
        out_shape=jax.ShapeDtypeStruct((1, D), jnp.float32),
    )(acc2, u2, dinv16, c_parts, b2.reshape(1, D), g2.reshape(1, D),
      be2.reshape(1, D), W3, b3.reshape(1, D))
    return out


# speedup vs baseline: 10.4851x; 10.4851x over previous
import functools

import jax
import jax.numpy as jnp
from jax import lax
from jax.experimental import pallas as pl
from jax.experimental.pallas import tpu as pltpu
from jax.experimental.pallas import tpu_sc as plsc

N = 10000
D = 128
EPS = 1e-5
L = 16
NC, NS = 2, 16
NW = NC * NS
K = 128
E = 320000
CPT = 80
IB = 16
NB = CPT // IB
EPAD = CPT * NW * K
NPAD = 10240
RPT = NPAD // NS

_mesh = plsc.VectorSubcoreMesh(core_axis_name="c", subcore_axis_name="s")
_sc_params = pltpu.CompilerParams(needs_layout_passes=False)


@functools.partial(
    pl.kernel,
    out_type=jax.ShapeDtypeStruct((NW, NPAD), jnp.float32),
    mesh=_mesh,
    compiler_params=_sc_params,
    scratch_types=[
        pltpu.VMEM((CPT, K), jnp.int32),
        pltpu.VMEM((NPAD,), jnp.float32),
    ],
)
def _deg_kernel(dst_hbm, out_hbm, dst_v, degp):
    cid = lax.axis_index("c")
    sid = lax.axis_index("s")
    wid = sid * NC + cid
    pltpu.sync_copy(dst_hbm.at[wid], dst_v)

    def zbody(i, carry):
        degp[pl.ds(i * L, L)] = jnp.zeros((L,), jnp.float32)
        return carry

    lax.fori_loop(0, NPAD // L, zbody, 0)

    def body(t, carry):
        j = t // (K // L)
        k = t % (K // L)
        idx = dst_v[j, pl.ds(k * L, L)]
        plsc.addupdate_scatter(degp, [idx], jnp.ones((L,), jnp.float32))
        return carry

    lax.fori_loop(0, CPT * (K // L), body, 0)
    pltpu.sync_copy(degp, out_hbm.at[wid])


def _make_edge_kernel(with_c):
    out_type = [jax.ShapeDtypeStruct((NC, NPAD, D), jnp.float32)]
    scratch = [
        pltpu.VMEM((IB, K), jnp.int32),
        pltpu.VMEM((IB, K), jnp.int32),
        pltpu.VMEM((K, D), jnp.float32),
        pltpu.VMEM_SHARED((NPAD, D), jnp.float32),
        pltpu.SemaphoreType.DMA,
    ]
    if with_c:
        out_type.append(jax.ShapeDtypeStruct((NW, NPAD), jnp.float32))
        scratch += [
            pltpu.VMEM((NPAD,), jnp.float32),
            pltpu.VMEM((NPAD,), jnp.float32),
        ]

    def kernel_body(*refs):
        if with_c:
            (src_hbm, dst_hbm, u_hbm, z128_hbm, dinv_hbm,
             acc_out, c_out,
             src_v, dst_v, rows_v, acc_sh, sem, dinv_v, cp) = refs
        else:
            (src_hbm, dst_hbm, u_hbm, z128_hbm,
             acc_out,
             src_v, dst_v, rows_v, acc_sh, sem) = refs
        cid = lax.axis_index("c")
        sid = lax.axis_index("s")
        wid = sid * NC + cid
        pltpu.sync_copy(z128_hbm, acc_sh.at[pl.ds(sid * RPT, RPT)])
        if with_c:
            pltpu.sync_copy(dinv_hbm, dinv_v)

            def czero(i, carry):
                cp[pl.ds(i * L, L)] = jnp.zeros((L,), jnp.float32)
                return carry

            lax.fori_loop(0, NPAD // L, czero, 0)
        plsc.subcore_barrier()

        def blk_body(b, carry):
            pltpu.sync_copy(src_hbm.at[wid, pl.ds(b * IB, IB)], src_v)
            pltpu.sync_copy(dst_hbm.at[wid, pl.ds(b * IB, IB)], dst_v)

            def body(j, carry2):
                pltpu.async_copy(u_hbm.at[src_v.at[j]], rows_v, sem).wait()
                pltpu.sync_copy(rows_v, acc_sh.at[dst_v.at[j]], add=True)
                if with_c:
                    def cbody(k, cc):
                        d16 = dst_v[j, pl.ds(k * L, L)]
                        s16 = src_v[j, pl.ds(k * L, L)]
                        vals = plsc.load_gather(dinv_v, [d16])
                        plsc.addupdate_scatter(cp, [s16], vals)
                        return cc

                    lax.fori_loop(0, K // L, cbody, 0)
                return carry2

            lax.fori_loop(0, IB, body, 0)
            return carry

        lax.fori_loop(0, NB, blk_body, 0)
        plsc.subcore_barrier()
        pltpu.sync_copy(acc_sh.at[pl.ds(sid * RPT, RPT)],
                        acc_out.at[cid, pl.ds(sid * RPT, RPT)])
        if with_c:
            pltpu.sync_copy(cp, c_out.at[wid])

    return pl.kernel(
        kernel_body,
        out_type=tuple(out_type) if with_c else out_type[0],
        mesh=_mesh,
        compiler_params=_sc_params,
        scratch_types=scratch,
    )


_edge_kernel_c = _make_edge_kernel(True)
_edge_kernel = _make_edge_kernel(False)


def _t1_body(degp_ref, x_ref, w1_ref, u1_ref, dinv_ref):
    deg = jnp.sum(degp_ref[...], axis=1, keepdims=True) + 1.0
    dinv = lax.rsqrt(deg)
    dinv_ref[...] = dinv
    h = jnp.dot(x_ref[...], w1_ref[...], preferred_element_type=jnp.float32)
    u1_ref[:N] = h * dinv[:N]
    u1_ref[N:] = jnp.zeros((NPAD - N, D), jnp.float32)


def _t2_body(acc_ref, u_ref, dinv_ref, b_ref, g_ref, be_ref, w2_ref, out_ref):
    dinv = dinv_ref[:N]
    z = (acc_ref[0, :N] + acc_ref[1, :N] + u_ref[:N]) * dinv + b_ref[...]
    m = jnp.mean(z, axis=0, keepdims=True)
    zc = z - m
    v = jnp.mean(zc * zc, axis=0, keepdims=True)
    h = jnp.maximum(zc * lax.rsqrt(v + EPS) * g_ref[...] + be_ref[...], 0.0)
    u2 = jnp.dot(h, w2_ref[...], preferred_element_type=jnp.float32) * dinv
    out_ref[:N] = u2
    out_ref[N:] = jnp.zeros((NPAD - N, D), jnp.float32)


def _t3_body(acc_ref, u_ref, dinv_ref, c_ref, b2_ref, g2_ref, be2_ref,
             w3_ref, b3_ref, out_ref):
    dinv = dinv_ref[:N]
    z = (acc_ref[0, :N] + acc_ref[1, :N] + u_ref[:N]) * dinv + b2_ref[...]
    m = jnp.mean(z, axis=0, keepdims=True)
    zc = z - m
    v = jnp.mean(zc * zc, axis=0, keepdims=True)
    h2 = jnp.maximum(zc * lax.rsqrt(v + EPS) * g2_ref[...] + be2_ref[...], 0.0)
    c = jnp.sum(c_ref[...], axis=1, keepdims=True)[:N]
    w = dinv * (c + dinv)
    s = jnp.sum(h2 * w, axis=0, keepdims=True)
    out_ref[...] = (jnp.dot(s, w3_ref[...],
                            preferred_element_type=jnp.float32) * (1.0 / N)
                    + b3_ref[...])


def kernel(x, edge_index, W1, b1, g1, be1, W2, b2, g2, be2, W3, b3):
    src = edge_index[0]
    dst = edge_index[1]
    padv = jnp.full((EPAD - E,), N, jnp.int32)
    srcp = jnp.concatenate([src, padv]).reshape(NW, CPT, K)
    dstp = jnp.concatenate([dst, padv]).reshape(NW, CPT, K)

    zeros128 = jnp.zeros((RPT, D), jnp.float32)

    deg_parts = _deg_kernel(dstp)

    u1, dinv2d = pl.pallas_call(
        _t1_body,
        out_shape=(jax.ShapeDtypeStruct((NPAD, D), jnp.float32),
                   jax.ShapeDtypeStruct((NPAD, 1), jnp.float32)),
    )(deg_parts.T, x, W1)
    dinv1d = dinv2d.reshape(NPAD)

    acc1, c_parts = _edge_kernel_c(srcp, dstp, u1, zeros128, dinv1d)

    u2 = pl.pallas_call(
        _t2_body,
        out_shape=jax.ShapeDtypeStruct((NPAD, D), jnp.float32),
    )(acc1, u1, dinv2d, b1.reshape(1, D), g1.reshape(1, D),
      be1.reshape(1, D), W2)

    acc2 = _edge_kernel(srcp, dstp, u2, zeros128)

    out = pl.pallas_call(
        _t3_body,
        out_shape=jax.ShapeDtypeStruct((1, D), jnp.float32),
    )(acc2, u2, dinv2d, c_parts.T, b2.reshape(1, D), g2.reshape(1, D),
      be2.reshape(1, D), W3, b3.reshape(1, D))
    return out


# speedup vs baseline: 11.9694x; 1.1416x over previous
import functools

import jax
import jax.numpy as jnp
from jax import lax
from jax.experimental import pallas as pl
from jax.experimental.pallas import tpu as pltpu
from jax.experimental.pallas import tpu_sc as plsc

N = 10000
D = 128
EPS = 1e-5
L = 16
NC, NS = 2, 16
NW = NC * NS
K = 128
E = 320000
CPT = 80
IB = 16
NB = CPT // IB
EPAD = CPT * NW * K
NPAD = 10240
RPT = NPAD // NS

_mesh = plsc.VectorSubcoreMesh(core_axis_name="c", subcore_axis_name="s")
_sc_params = pltpu.CompilerParams(needs_layout_passes=False)


@functools.partial(
    pl.kernel,
    out_type=jax.ShapeDtypeStruct((NW, NPAD), jnp.float32),
    mesh=_mesh,
    compiler_params=_sc_params,
    scratch_types=[
        pltpu.VMEM((CPT, K), jnp.int32),
        pltpu.VMEM((NPAD,), jnp.float32),
    ],
)
def _deg_kernel(dst_hbm, out_hbm, dst_v, degp):
    cid = lax.axis_index("c")
    sid = lax.axis_index("s")
    wid = sid * NC + cid
    pltpu.sync_copy(dst_hbm.at[wid], dst_v)

    def zbody(i, carry):
        degp[pl.ds(i * L, L)] = jnp.zeros((L,), jnp.float32)
        return carry

    lax.fori_loop(0, NPAD // L, zbody, 0)

    def body(t, carry):
        j = t // (K // L)
        k = t % (K // L)
        idx = dst_v[j, pl.ds(k * L, L)]
        plsc.addupdate_scatter(degp, [idx], jnp.ones((L,), jnp.float32))
        return carry

    lax.fori_loop(0, CPT * (K // L), body, 0)
    pltpu.sync_copy(degp, out_hbm.at[wid])


@functools.partial(
    pl.kernel,
    out_type=jax.ShapeDtypeStruct((NC, NPAD, D), jnp.float32),
    mesh=_mesh,
    compiler_params=_sc_params,
    scratch_types=[
        pltpu.VMEM((IB, K), jnp.int32),
        pltpu.VMEM((IB, K), jnp.int32),
        pltpu.VMEM((2, K, D), jnp.float32),
        pltpu.VMEM_SHARED((NPAD, D), jnp.float32),
        pltpu.SemaphoreType.DMA,
        pltpu.SemaphoreType.DMA,
    ],
)
def _edge_kernel(src_hbm, dst_hbm, u_hbm, z128_hbm, acc_out,
                 src_v, dst_v, rows_v, acc_sh, sem0, sem1):
    cid = lax.axis_index("c")
    sid = lax.axis_index("s")
    wid = sid * NC + cid
    pltpu.sync_copy(z128_hbm, acc_sh.at[pl.ds(sid * RPT, RPT)])
    plsc.subcore_barrier()

    def blk_body(b, carry):
        pltpu.sync_copy(src_hbm.at[wid, pl.ds(b * IB, IB)], src_v)
        pltpu.sync_copy(dst_hbm.at[wid, pl.ds(b * IB, IB)], dst_v)
        pltpu.async_copy(u_hbm.at[src_v.at[0]], rows_v.at[0], sem0)

        def pair(p, carry2):
            j0 = 2 * p
            j1 = j0 + 1
            pltpu.async_copy(u_hbm.at[src_v.at[j1]], rows_v.at[1], sem1)
            pltpu.make_async_copy(
                u_hbm.at[src_v.at[j0]], rows_v.at[0], sem0).wait()
            pltpu.sync_copy(rows_v.at[0], acc_sh.at[dst_v.at[j0]], add=True)

            @pl.when(j1 + 1 < IB)
            def _():
                pltpu.async_copy(u_hbm.at[src_v.at[j1 + 1]],
                                 rows_v.at[0], sem0)

            pltpu.make_async_copy(
                u_hbm.at[src_v.at[j1]], rows_v.at[1], sem1).wait()
            pltpu.sync_copy(rows_v.at[1], acc_sh.at[dst_v.at[j1]], add=True)
            return carry2

        lax.fori_loop(0, IB // 2, pair, 0)
        return carry

    lax.fori_loop(0, NB, blk_body, 0)
    plsc.subcore_barrier()
    pltpu.sync_copy(acc_sh.at[pl.ds(sid * RPT, RPT)],
                    acc_out.at[cid, pl.ds(sid * RPT, RPT)])


@functools.partial(
    pl.kernel,
    out_type=jax.ShapeDtypeStruct((NW, NPAD), jnp.float32),
    mesh=_mesh,
    compiler_params=_sc_params,
    scratch_types=[
        pltpu.VMEM((CPT, K), jnp.int32),
        pltpu.VMEM((CPT, K), jnp.int32),
        pltpu.VMEM((NPAD,), jnp.float32),
        pltpu.VMEM((NPAD,), jnp.float32),
    ],
)
def _c_kernel(src_hbm, dst_hbm, dinv_hbm, out_hbm, src_v, dst_v, dinv_v, cp):
    cid = lax.axis_index("c")
    sid = lax.axis_index("s")
    wid = sid * NC + cid
    pltpu.sync_copy(src_hbm.at[wid], src_v)
    pltpu.sync_copy(dst_hbm.at[wid], dst_v)
    pltpu.sync_copy(dinv_hbm, dinv_v)

    def czero(i, carry):
        cp[pl.ds(i * L, L)] = jnp.zeros((L,), jnp.float32)
        return carry

    lax.fori_loop(0, NPAD // L, czero, 0)

    def body(t, carry):
        j = t // (K // L)
        k = t % (K // L)
        d16 = dst_v[j, pl.ds(k * L, L)]
        s16 = src_v[j, pl.ds(k * L, L)]
        vals = plsc.load_gather(dinv_v, [d16])
        plsc.addupdate_scatter(cp, [s16], vals)
        return carry

    lax.fori_loop(0, CPT * (K // L), body, 0)
    pltpu.sync_copy(cp, out_hbm.at[wid])


def _t1_body(degp_ref, x_ref, w1_ref, u1_ref, dinv_ref):
    deg = jnp.sum(degp_ref[...], axis=1, keepdims=True) + 1.0
    dinv = lax.rsqrt(deg)
    dinv_ref[...] = dinv
    h = jnp.dot(x_ref[...], w1_ref[...], preferred_element_type=jnp.float32)
    u1_ref[:N] = h * dinv[:N]
    u1_ref[N:] = jnp.zeros((NPAD - N, D), jnp.float32)


def _t2_body(acc_ref, u_ref, dinv_ref, b_ref, g_ref, be_ref, w2_ref, out_ref):
    dinv = dinv_ref[:N]
    z = (acc_ref[0, :N] + acc_ref[1, :N] + u_ref[:N]) * dinv + b_ref[...]
    m = jnp.mean(z, axis=0, keepdims=True)
    zc = z - m
    v = jnp.mean(zc * zc, axis=0, keepdims=True)
    h = jnp.maximum(zc * lax.rsqrt(v + EPS) * g_ref[...] + be_ref[...], 0.0)
    u2 = jnp.dot(h, w2_ref[...], preferred_element_type=jnp.float32) * dinv
    out_ref[:N] = u2
    out_ref[N:] = jnp.zeros((NPAD - N, D), jnp.float32)


def _t3_body(acc_ref, u_ref, dinv_ref, c_ref, b2_ref, g2_ref, be2_ref,
             w3_ref, b3_ref, out_ref):
    dinv = dinv_ref[:N]
    z = (acc_ref[0, :N] + acc_ref[1, :N] + u_ref[:N]) * dinv + b2_ref[...]
    m = jnp.mean(z, axis=0, keepdims=True)
    zc = z - m
    v = jnp.mean(zc * zc, axis=0, keepdims=True)
    h2 = jnp.maximum(zc * lax.rsqrt(v + EPS) * g2_ref[...] + be2_ref[...], 0.0)
    c = jnp.sum(c_ref[...], axis=1, keepdims=True)[:N]
    w = dinv * (c + dinv)
    s = jnp.sum(h2 * w, axis=0, keepdims=True)
    out_ref[...] = (jnp.dot(s, w3_ref[...],
                            preferred_element_type=jnp.float32) * (1.0 / N)
                    + b3_ref[...])


def kernel(x, edge_index, W1, b1, g1, be1, W2, b2, g2, be2, W3, b3):
    src = edge_index[0]
    dst = edge_index[1]
    padv = jnp.full((EPAD - E,), N, jnp.int32)
    srcp = jnp.concatenate([src, padv]).reshape(NW, CPT, K)
    dstp = jnp.concatenate([dst, padv]).reshape(NW, CPT, K)

    zeros128 = jnp.zeros((RPT, D), jnp.float32)

    deg_parts = _deg_kernel(dstp)

    u1, dinv2d = pl.pallas_call(
        _t1_body,
        out_shape=(jax.ShapeDtypeStruct((NPAD, D), jnp.float32),
                   jax.ShapeDtypeStruct((NPAD, 1), jnp.float32)),
    )(deg_parts.T, x, W1)
    dinv1d = dinv2d.reshape(NPAD)

    acc1 = _edge_kernel(srcp, dstp, u1, zeros128)
    c_parts = _c_kernel(srcp, dstp, dinv1d)

    u2 = pl.pallas_call(
        _t2_body,
        out_shape=jax.ShapeDtypeStruct((NPAD, D), jnp.float32),
    )(acc1, u1, dinv2d, b1.reshape(1, D), g1.reshape(1, D),
      be1.reshape(1, D), W2)

    acc2 = _edge_kernel(srcp, dstp, u2, zeros128)

    out = pl.pallas_call(
        _t3_body,
        out_shape=jax.ShapeDtypeStruct((1, D), jnp.float32),
    )(acc2, u2, dinv2d, c_parts.T, b2.reshape(1, D), g2.reshape(1, D),
      be2.reshape(1, D), W3, b3.reshape(1, D))
    return out
